# baseline (speedup 1.0000x reference)
import functools

import jax
import jax.numpy as jnp
from jax import lax
from jax.experimental import pallas as pl
from jax.experimental.pallas import tpu as pltpu
from jax.experimental.pallas import tpu_sc as plsc

P = 10000
T = 10000
E = 320000
D = 128
W = D // 2
NC, NS, L = 2, 16, 16
NW = NC * NS

PK_SUB = T // NS

P_PAD = 10240
HP_SUB = P_PAD // NS
RSUB = 80

EW = E // NW
EC = 80
NCHUNK = EW // EC
NBUF = 2


@functools.partial(
    pl.kernel,
    mesh=plsc.VectorSubcoreMesh(core_axis_name="c", subcore_axis_name="s"),
    compiler_params=pltpu.CompilerParams(needs_layout_passes=False,
                                         use_tc_tiling_on_sc=False,
                                         internal_scratch_in_bytes=4096),
    out_type=jax.ShapeDtypeStruct((E,), jnp.float32),
    scratch_types=[
        pltpu.VMEM_SHARED((T, W), jnp.int32),
        pltpu.VMEM_SHARED((P_PAD, W), jnp.int32),
        pltpu.VMEM((HP_SUB,), jnp.int32),
        pltpu.VMEM((HP_SUB,), jnp.int32),
        pltpu.VMEM((EW,), jnp.int32),
        pltpu.VMEM((EW,), jnp.int32),
        *([pltpu.VMEM((EC, W), jnp.int32)] * 4),
        *([pltpu.VMEM((EC,), jnp.float32)] * 2),
        *([pltpu.SemaphoreType.DMA] * 8),
    ],
)
def _fused_kernel(emb, s0, s1, src, dst, out,
                  emb_s, hp_s, i0_v, i1_v, src_v, dst_v,
                  a0, a1, b0, b1, so0, so1,
                  sa0, sa1, sb0, sb1, so_s0, so_s1, spk0, spk1):
    sid = lax.axis_index("s")
    cid = lax.axis_index("c")
    wid = sid * NC + cid

    eb = wid * EW
    hb = sid * HP_SUB
    h_src = pltpu.async_copy(src.at[pl.ds(eb, EW)], src_v, so_s0)
    h_dst = pltpu.async_copy(dst.at[pl.ds(eb, EW)], dst_v, so_s1)
    h_s0 = pltpu.async_copy(s0.at[pl.ds(hb, HP_SUB)], i0_v, sa0)
    h_s1 = pltpu.async_copy(s1.at[pl.ds(hb, HP_SUB)], i1_v, sb0)

    PK_SZ = [31] * 20 + [5]
    PK_OFF = [31 * i for i in range(21)]

    def pack_phase(pf0, pf1):
        pfs, pk_sems = (pf0, pf1), (spk0, spk1)
        hs = {}

        def pk_issue(ci):
            u, sz = ci % 2, PK_SZ[ci]
            hs[ci] = pltpu.async_copy(
                emb.at[pl.ds(sid * PK_SUB + PK_OFF[ci], sz)],
                pfs[u].at[pl.ds(0, sz)], pk_sems[u])

        pk_issue(0)
        pk_issue(1)
        for ci, sz in enumerate(PK_SZ):
            hs[ci].wait()
            pf = pfs[ci % 2]

            def prow(r, _):
                for q in range(D // (2 * L)):
                    pair = plsc.pack(
                        pf[r, pl.ds(q * 2 * L, L)],
                        pf[r, pl.ds(q * 2 * L + L, L)],
                        format=plsc.PackFormat.INTERLEAVED)
                    b0[r, pl.ds(q * L, L)] = plsc.bitcast(pair, jnp.int32)
                return 0

            lax.fori_loop(0, sz, prow, 0)
            pltpu.sync_copy(b0.at[pl.ds(0, sz)],
                            emb_s.at[pl.ds(sid * PK_SUB + PK_OFF[ci], sz)])
            if ci + 2 < len(PK_SZ):
                pk_issue(ci + 2)

    pl.run_scoped(pack_phase,
                  pltpu.VMEM((31, D), jnp.float32),
                  pltpu.VMEM((31, D), jnp.float32))
    plsc.subcore_barrier()

    h_s0.wait()
    h_s1.wait()
    half = jnp.full((2 * L,), 0.5, jnp.bfloat16)
    HP_N = HP_SUB // RSUB
    hp_a, hp_b = (a0, a1), (b0, b1)
    hp_sa, hp_sb = (sa0, sa1), (sb0, sb1)
    hps = {}

    def hp_issue(k):
        u = k % 2
        hps[k] = (
            pltpu.async_copy(emb_s.at[i0_v.at[pl.ds(k * RSUB, RSUB)]],
                             hp_a[u], hp_sa[u]),
            pltpu.async_copy(emb_s.at[i1_v.at[pl.ds(k * RSUB, RSUB)]],
                             hp_b[u], hp_sb[u]),
        )

    hp_issue(0)
    hp_issue(1)
    for k in range(HP_N):
        ca, cb = hps[k]
        ca.wait()
        cb.wait()
        av, bv = hp_a[k % 2], hp_b[k % 2]

        def hrow(r, _):
            for q in range(W // L):
                sl = pl.ds(q * L, L)
                m = (plsc.bitcast(av[r, sl], jnp.bfloat16) +
                     plsc.bitcast(bv[r, sl], jnp.bfloat16)) * half
                av[r, sl] = plsc.bitcast(m, jnp.int32)
            return 0

        lax.fori_loop(0, RSUB, hrow, 0)
        pltpu.sync_copy(av, hp_s.at[pl.ds(hb + k * RSUB, RSUB)])
        if k + 2 < HP_N:
            hp_issue(k + 2)
    plsc.subcore_barrier()

    h_src.wait()
    h_dst.wait()

    a_bufs, b_bufs = (a0, a1), (b0, b1)
    a_sems, b_sems = (sa0, sa1), (sb0, sb1)
    so_bufs, so_sems = (so0, so1), (so_s0, so_s1)

    def idx_a(c):
        return src_v.at[pl.ds(pl.multiple_of(c * EC, 8), EC)]

    def idx_b(c):
        return dst_v.at[pl.ds(pl.multiple_of(c * EC, 8), EC)]

    def out_at(c):
        return out.at[pl.ds(eb + pl.multiple_of(c * EC, 8), EC)]

    def issue(c, u):
        pltpu.async_copy(hp_s.at[idx_a(c)], a_bufs[u], a_sems[u])
        pltpu.async_copy(emb_s.at[idx_b(c)], b_bufs[u], b_sems[u])

    def wait(c, u):
        pltpu.make_async_copy(hp_s.at[idx_a(c)], a_bufs[u], a_sems[u]).wait()
        pltpu.make_async_copy(emb_s.at[idx_b(c)], b_bufs[u], b_sems[u]).wait()

    def compute(c, u):
        a_v, b_v = a_bufs[u], b_bufs[u]
        lane = lax.iota(jnp.int32, L)
        for g in range(EC // L):
            rows = lane + g * L
            acc0 = jnp.zeros((L,), jnp.float32)
            acc1 = jnp.zeros((L,), jnp.float32)

            def wstep(w8, accs):
                acc0, acc1 = accs
                for uu in range(8):
                    cols = (lane + (w8 * 8 + uu)) & (W - 1)
                    wa = plsc.load_gather(a_v, [rows, cols])
                    wb = plsc.load_gather(b_v, [rows, cols])
                    wp = (plsc.bitcast(wa, jnp.bfloat16) *
                          plsc.bitcast(wb, jnp.bfloat16))
                    p_lo, p_hi = plsc.unpack(
                        wp, format=plsc.PackFormat.INTERLEAVED)
                    acc0 = acc0 + p_lo
                    acc1 = acc1 + p_hi
                return acc0, acc1

            acc0, acc1 = lax.fori_loop(0, W // 8, wstep, (acc0, acc1))
            so_bufs[u][pl.ds(g * L, L)] = acc0 + acc1

    def body(c, u, static):
        wait(c, u)
        if static:
            if c >= NBUF:
                pltpu.make_async_copy(so_bufs[u], out_at(c - NBUF),
                                      so_sems[u]).wait()
        else:
            @pl.when(c >= NBUF)
            def _():
                pltpu.make_async_copy(so_bufs[u], out_at(c - NBUF),
                                      so_sems[u]).wait()
        compute(c, u)
        pltpu.async_copy(so_bufs[u], out_at(c), so_sems[u])
        if static:
            if c + NBUF < NCHUNK:
                issue(c + NBUF, u)
        else:
            @pl.when(c + NBUF < NCHUNK)
            def _():
                issue(c + NBUF, u)

    for j in range(NBUF):
        issue(j, j)

    FI = NCHUNK // NBUF - 1

    def ring(i2, _):
        for u in range(NBUF):
            body(i2 * NBUF + u, u, static=False)
        return 0

    lax.fori_loop(0, FI, ring, 0)
    for c in range(FI * NBUF, NCHUNK):
        body(c, c % NBUF, static=True)
    for u in range(NBUF):
        c_last = ((NCHUNK - 1 - u) // NBUF) * NBUF + u
        pltpu.make_async_copy(so_bufs[u], out_at(c_last), so_sems[u]).wait()


def kernel(track_emb, edge_index, sampled_tracks):
    track_emb = track_emb.astype(jnp.float32)
    src = edge_index[0].astype(jnp.int32)
    dst = edge_index[1].astype(jnp.int32)
    st = sampled_tracks.astype(jnp.int32)
    s0 = jnp.pad(st[:, 0], (0, P_PAD - P))
    s1 = jnp.pad(st[:, 1], (0, P_PAD - P))
    return _fused_kernel(track_emb, s0, s1, src, dst)
